# baseline (speedup 1.0000x reference)
import functools

import jax
import jax.numpy as jnp
from jax import lax
from jax.experimental import pallas as pl
from jax.experimental.pallas import tpu as pltpu
from jax.experimental.pallas import tpu_sc as plsc

NC, NS, LANES = 2, 16, 16
NW = NC * NS


def _sc_mesh():
    return plsc.VectorSubcoreMesh(
        core_axis_name="c", subcore_axis_name="s", num_cores=NC, num_subcores=NS)


def _perm128():
    g = jnp.arange(128, dtype=jnp.int32)
    grp, k = g // 32, g % 32
    return jnp.where(k < 16, grp * 32 + 2 * k, grp * 32 + 2 * (k - 16) + 1)


def _sc_degree(dst_p, ew_p, z_col, n_pad):
    ep = dst_p.shape[0]
    pt = ep // NW
    ch = 512
    n_ch = pt // ch
    rpt = n_pad // NS

    @functools.partial(
        pl.kernel,
        out_type=jax.ShapeDtypeStruct((NC, n_pad), jnp.float32),
        mesh=_sc_mesh(),
        scratch_types=[
            pltpu.VMEM((ch,), jnp.int32),
            pltpu.VMEM((ch,), jnp.float32),
            pltpu.VMEM_SHARED((n_pad,), jnp.float32),
        ],
    )
    def deg_kernel(dst_hbm, ew_hbm, z_hbm, out_hbm, dstv, ewv, acc):
        cid = lax.axis_index("c")
        sid = lax.axis_index("s")
        wid = sid * NC + cid
        pltpu.sync_copy(z_hbm, acc.at[pl.ds(sid * rpt, rpt)])
        plsc.subcore_barrier()

        def body(i, carry):
            base = wid * pt + i * ch
            pltpu.sync_copy(dst_hbm.at[pl.ds(base, ch)], dstv)
            pltpu.sync_copy(ew_hbm.at[pl.ds(base, ch)], ewv)
            pltpu.sync_copy(ewv, acc.at[dstv], add=True)
            return carry

        lax.fori_loop(0, n_ch, body, 0)
        plsc.subcore_barrier()
        pltpu.sync_copy(acc.at[pl.ds(sid * rpt, rpt)],
                        out_hbm.at[cid, pl.ds(sid * rpt, rpt)])

    return deg_kernel(dst_p, ew_p, z_col)


def _sc_propagate(src_p, dst_p, ew_p, hpack, n_pad, d):
    ep = src_p.shape[0]
    ch = 96
    nck = ep // ch
    d2 = d // 2
    m0 = (nck * 8 // 10) // NS // 2 * 2
    m1 = nck // NS - m0
    g16 = ch // LANES
    rpt = n_pad // NS
    msk = jnp.int32(-65536)

    @functools.partial(
        pl.kernel,
        out_type=jax.ShapeDtypeStruct((NC, n_pad, d), jnp.float32),
        mesh=_sc_mesh(),
        compiler_params=pltpu.CompilerParams(use_tc_tiling_on_sc=False,
                                             needs_layout_passes=False),
        scratch_types=[
            pltpu.VMEM((ch,), jnp.int32),
            pltpu.VMEM((ch,), jnp.int32),
            pltpu.VMEM((ch,), jnp.float32),
            pltpu.VMEM((ch,), jnp.int32),
            pltpu.VMEM((ch,), jnp.int32),
            pltpu.VMEM((ch,), jnp.float32),
            pltpu.VMEM((ch, d2), jnp.int32),
            pltpu.VMEM((ch, d2), jnp.int32),
            pltpu.VMEM((ch, d), jnp.float32),
            pltpu.VMEM((ch, d), jnp.float32),
            pltpu.VMEM_SHARED((n_pad, d), jnp.float32),
            pltpu.SemaphoreType.DMA,
            pltpu.SemaphoreType.DMA,
            pltpu.SemaphoreType.DMA,
            pltpu.SemaphoreType.DMA,
        ],
    )
    def prop_kernel(src_hbm, dst_hbm, ew_hbm, h_hbm, out_hbm,
                    srcv0, dstv0, ewv0, srcv1, dstv1, ewv1,
                    gin0, gin1, rows0, rows1, acc, g0, g1, s0, s1):
        cid = lax.axis_index("c")
        sid = lax.axis_index("s")
        start = jnp.where(cid == 0, sid * m0, NS * m0 + sid * m1)
        count = jnp.where(cid == 0, m0, m1)

        def zfill(r, c0):
            for j in range(d // LANES):
                rows0[r, pl.ds(j * LANES, LANES)] = jnp.zeros(
                    (LANES,), jnp.float32)
            return c0

        lax.fori_loop(0, ch, zfill, 0)
        nzc = rpt // ch
        for j in range(nzc):
            pltpu.sync_copy(rows0, acc.at[pl.ds(sid * rpt + j * ch, ch)])
        if rpt % ch:
            pltpu.sync_copy(rows0.at[pl.ds(0, rpt % ch)],
                            acc.at[pl.ds(sid * rpt + nzc * ch, rpt % ch)])
        plsc.subcore_barrier()

        def scale(gin, rows, ewv):
            def group(g, c2):
                ew_vec = ewv[pl.ds(g * LANES, LANES)]
                for l in range(LANES):
                    lane = jnp.full((LANES, 1), l, jnp.int32)
                    ew_b = lax.gather(
                        ew_vec, lane,
                        lax.GatherDimensionNumbers(
                            offset_dims=(), collapsed_slice_dims=(0,),
                            start_index_map=(0,)),
                        (1,), mode=lax.GatherScatterMode.PROMISE_IN_BOUNDS)
                    row = g * LANES + l
                    for j in range(d2 // LANES):
                        xi = gin[row, pl.ds(j * LANES, LANES)]
                        lo = plsc.bitcast(xi << 16, jnp.float32)
                        hi = plsc.bitcast(xi & msk, jnp.float32)
                        rows[row, pl.ds(j * 2 * LANES, LANES)] = lo * ew_b
                        rows[row, pl.ds((2 * j + 1) * LANES, LANES)] = (
                            hi * ew_b)
                return c2
            lax.fori_loop(0, g16, group, 0)

        def load_idx(i, srcv, dstv, ewv):
            base = (start + i) * ch
            pltpu.sync_copy(src_hbm.at[pl.ds(base, ch)], srcv)
            pltpu.sync_copy(dst_hbm.at[pl.ds(base, ch)], dstv)
            pltpu.sync_copy(ew_hbm.at[pl.ds(base, ch)], ewv)

        load_idx(0, srcv0, dstv0, ewv0)
        pltpu.async_copy(h_hbm.at[srcv0], gin0, g0)

        def pair(k, carry):
            a = 2 * k
            b = a + 1

            @pl.when(k > 0)
            def _():
                pltpu.make_async_copy(rows1, acc.at[dstv1], s1).wait()

            load_idx(b, srcv1, dstv1, ewv1)
            pltpu.async_copy(h_hbm.at[srcv1], gin1, g1)
            pltpu.make_async_copy(h_hbm.at[srcv0], gin0, g0).wait()
            scale(gin0, rows0, ewv0)
            pltpu.async_copy(rows0, acc.at[dstv0], s0, add=True)
            pltpu.make_async_copy(h_hbm.at[srcv1], gin1, g1).wait()
            scale(gin1, rows1, ewv1)
            pltpu.async_copy(rows1, acc.at[dstv1], s1, add=True)

            @pl.when(k < count // 2 - 1)
            def _():
                pltpu.make_async_copy(rows0, acc.at[dstv0], s0).wait()
                load_idx(a + 2, srcv0, dstv0, ewv0)
                pltpu.async_copy(h_hbm.at[srcv0], gin0, g0)

            return carry

        lax.fori_loop(0, count // 2, pair, 0)
        pltpu.make_async_copy(rows0, acc.at[dstv0], s0).wait()
        pltpu.make_async_copy(rows1, acc.at[dstv1], s1).wait()
        plsc.subcore_barrier()
        pltpu.sync_copy(acc.at[pl.ds(sid * rpt, rpt)],
                        out_hbm.at[cid, pl.ds(sid * rpt, rpt)])

    return prop_kernel(src_p, dst_p, ew_p, hpack)


def _tc_first(x, w1, deg0, deg1, bn):
    n, f = x.shape
    d = w1.shape[1]
    grid = n // bn

    def body(x_ref, w_ref, d0_ref, d1_ref, h_ref, dinv_ref):
        deg = d0_ref[...] + d1_ref[...] + 1.0
        dinv = lax.rsqrt(deg)
        dinv_ref[...] = dinv
        h = jnp.dot(x_ref[...], w_ref[...], preferred_element_type=jnp.float32)
        h_ref[...] = h * dinv

    return pl.pallas_call(
        body,
        grid=(grid,),
        in_specs=[
            pl.BlockSpec((bn, f), lambda i: (i, 0)),
            pl.BlockSpec((f, d), lambda i: (0, 0)),
            pl.BlockSpec((bn, 1), lambda i: (i, 0)),
            pl.BlockSpec((bn, 1), lambda i: (i, 0)),
        ],
        out_specs=[
            pl.BlockSpec((bn, d), lambda i: (i, 0)),
            pl.BlockSpec((bn, 1), lambda i: (i, 0)),
        ],
        out_shape=[
            jax.ShapeDtypeStruct((n, d), jnp.float32),
            jax.ShapeDtypeStruct((n, 1), jnp.float32),
        ],
    )(x, w1, deg0, deg1)


def _tc_combine_matmul(acc, dinv, b_row, w, bn):
    n = dinv.shape[0]
    d = acc.shape[2]
    d2 = w.shape[1]
    grid = n // bn

    def body(a_ref, dinv_ref, b_ref, w_ref, o_ref):
        dinv = dinv_ref[...]
        z = dinv * (a_ref[0] + a_ref[1]) + b_ref[...]
        a = jnp.maximum(z, 0.0)
        o_ref[...] = dinv * jnp.dot(a, w_ref[...],
                                    preferred_element_type=jnp.float32)

    return pl.pallas_call(
        body,
        grid=(grid,),
        in_specs=[
            pl.BlockSpec((NC, bn, d), lambda i: (0, i, 0)),
            pl.BlockSpec((bn, 1), lambda i: (i, 0)),
            pl.BlockSpec((1, d), lambda i: (0, 0)),
            pl.BlockSpec((d, d2), lambda i: (0, 0)),
        ],
        out_specs=pl.BlockSpec((bn, d2), lambda i: (i, 0)),
        out_shape=jax.ShapeDtypeStruct((n, d2), jnp.float32),
    )(acc, dinv, b_row, w)


def _tc_pool_mlp(acc, dinv, b_row, batch3, wl1, bl1_row, wl2, bl2_row,
                 n_graphs, bn):
    n = dinv.shape[0]
    d = acc.shape[2]
    grid = n // bn
    dm = wl1.shape[1]
    c = wl2.shape[1]

    def body(a_ref, dinv_ref, b_ref, bt_ref, wl1_ref, bl1_ref,
             wl2_ref, bl2_ref, o_ref, sums, cnts):
        i = pl.program_id(0)
        z = dinv_ref[...] * (a_ref[0] + a_ref[1]) + b_ref[...]
        a = jnp.maximum(z, 0.0)
        bt = bt_ref[...].reshape(1, bn)
        gids = lax.broadcasted_iota(jnp.int32, (n_graphs, bn), 0)
        p = jnp.where(gids == bt, 1.0, 0.0)

        @pl.when(i == 0)
        def _():
            sums[...] = jnp.zeros_like(sums)
            cnts[...] = jnp.zeros_like(cnts)

        sums[...] += jnp.dot(p, a, preferred_element_type=jnp.float32)
        cnts[...] += jnp.sum(p, axis=1, keepdims=True)

        @pl.when(i == grid - 1)
        def _():
            pooled = sums[...] / jnp.maximum(cnts[...], 1.0)
            hm = jnp.maximum(
                jnp.dot(pooled, wl1_ref[...],
                        preferred_element_type=jnp.float32) + bl1_ref[...], 0.0)
            o_ref[...] = jnp.dot(hm, wl2_ref[...],
                                 preferred_element_type=jnp.float32) + bl2_ref[...]

    return pl.pallas_call(
        body,
        grid=(grid,),
        in_specs=[
            pl.BlockSpec((NC, bn, d), lambda i: (0, i, 0)),
            pl.BlockSpec((bn, 1), lambda i: (i, 0)),
            pl.BlockSpec((1, d), lambda i: (0, 0)),
            pl.BlockSpec((1, 1, bn), lambda i: (i, 0, 0)),
            pl.BlockSpec((d, dm), lambda i: (0, 0)),
            pl.BlockSpec((1, dm), lambda i: (0, 0)),
            pl.BlockSpec((dm, c), lambda i: (0, 0)),
            pl.BlockSpec((1, c), lambda i: (0, 0)),
        ],
        out_specs=pl.BlockSpec((n_graphs, c), lambda i: (0, 0)),
        out_shape=jax.ShapeDtypeStruct((n_graphs, c), jnp.float32),
        scratch_shapes=[
            pltpu.VMEM((n_graphs, d), jnp.float32),
            pltpu.VMEM((n_graphs, 1), jnp.float32),
        ],
    )(acc, dinv, b_row, batch3, wl1, bl1_row, wl2, bl2_row)


def _pack(h):
    n, d = h.shape
    return lax.bitcast_convert_type(
        h.astype(jnp.bfloat16).reshape(n, d // 2, 2), jnp.int32)


def kernel(x, edge_index, edge_attr, batch, W1, b1, W2, b2, W3, b3,
           Wl1, bl1, Wl2, bl2):
    n, f = x.shape
    e = edge_index.shape[1]
    g = 64
    bn = 1000
    n_pad = 10240
    d_all = W2.shape[1]

    src = edge_index[0].astype(jnp.int32)
    dst = edge_index[1].astype(jnp.int32)
    ew = edge_attr.astype(jnp.float32)

    epd = 163840
    dst_d = jnp.concatenate([dst, jnp.zeros((epd - e,), jnp.int32)])
    ew_d = jnp.concatenate([ew, jnp.zeros((epd - e,), jnp.float32)])

    loop = jnp.arange(n, dtype=jnp.int32)
    epp = 172032
    padp = epp - e - n
    src_p = jnp.concatenate([src, loop, jnp.zeros((padp,), jnp.int32)])
    dst_p = jnp.concatenate([dst, loop, jnp.zeros((padp,), jnp.int32)])
    ew_p = jnp.concatenate([ew, jnp.ones((n,), jnp.float32),
                            jnp.zeros((padp,), jnp.float32)])

    w1p = jnp.pad(W1, ((0, 0), (0, d_all - W1.shape[1])))
    b1p = jnp.pad(b1, (0, d_all - b1.shape[0]))
    w2p = jnp.pad(W2, ((0, d_all - W2.shape[0]), (0, 0)))

    pidx = _perm128()
    b1s = b1p[pidx].reshape(1, -1)
    w2s = w2p[pidx, :]
    b2s = b2[pidx].reshape(1, -1)
    w3s = W3[pidx, :]
    b3s = b3[pidx].reshape(1, -1)
    wl1s = Wl1[pidx, :]

    z_col = jnp.zeros((n_pad // NS,), jnp.float32)

    deg = _sc_degree(dst_d, ew_d, z_col, n_pad)
    deg0 = deg[0, :n].reshape(n, 1)
    deg1 = deg[1, :n].reshape(n, 1)

    h1s, dinv = _tc_first(x, w1p, deg0, deg1, bn)
    acc1 = _sc_propagate(src_p, dst_p, ew_p, _pack(h1s), n_pad, d_all)

    h2s = _tc_combine_matmul(acc1, dinv, b1s, w2s, bn)
    acc2 = _sc_propagate(src_p, dst_p, ew_p, _pack(h2s), n_pad, d_all)

    h3s = _tc_combine_matmul(acc2, dinv, b2s, w3s, bn)
    acc3 = _sc_propagate(src_p, dst_p, ew_p, _pack(h3s), n_pad, d_all)

    batch3 = batch.astype(jnp.int32).reshape(n // bn, 1, bn)
    out = _tc_pool_mlp(acc3, dinv, b3s, batch3,
                       wl1s, bl1.reshape(1, -1), Wl2, bl2.reshape(1, -1),
                       g, bn)
    return out


# speedup vs baseline: 1.2545x; 1.2545x over previous
import functools

import jax
import jax.numpy as jnp
from jax import lax
from jax.experimental import pallas as pl
from jax.experimental.pallas import tpu as pltpu
from jax.experimental.pallas import tpu_sc as plsc

NC, NS, LANES = 2, 16, 16
NW = NC * NS


def _sc_mesh():
    return plsc.VectorSubcoreMesh(
        core_axis_name="c", subcore_axis_name="s", num_cores=NC, num_subcores=NS)


def _sc_degree(dst_p, ew_p, z_col, n_pad):
    ep = dst_p.shape[0]
    pt = ep // NW
    ch = 512
    n_ch = pt // ch
    rpt = n_pad // NS

    @functools.partial(
        pl.kernel,
        out_type=jax.ShapeDtypeStruct((NC, n_pad), jnp.float32),
        mesh=_sc_mesh(),
        scratch_types=[
            pltpu.VMEM((ch,), jnp.int32),
            pltpu.VMEM((ch,), jnp.float32),
            pltpu.VMEM_SHARED((n_pad,), jnp.float32),
        ],
    )
    def deg_kernel(dst_hbm, ew_hbm, z_hbm, out_hbm, dstv, ewv, acc):
        cid = lax.axis_index("c")
        sid = lax.axis_index("s")
        wid = sid * NC + cid
        pltpu.sync_copy(z_hbm, acc.at[pl.ds(sid * rpt, rpt)])
        plsc.subcore_barrier()

        def body(i, carry):
            base = wid * pt + i * ch
            pltpu.sync_copy(dst_hbm.at[pl.ds(base, ch)], dstv)
            pltpu.sync_copy(ew_hbm.at[pl.ds(base, ch)], ewv)
            pltpu.sync_copy(ewv, acc.at[dstv], add=True)
            return carry

        lax.fori_loop(0, n_ch, body, 0)
        plsc.subcore_barrier()
        pltpu.sync_copy(acc.at[pl.ds(sid * rpt, rpt)],
                        out_hbm.at[cid, pl.ds(sid * rpt, rpt)])

    return deg_kernel(dst_p, ew_p, z_col)


def _sc_propagate(src3, dst3, ew_p, h, z_blk, nch, n_pad, d):
    ep = nch * 256
    src_p = src3.reshape(-1)[:ep]
    dst_p = dst3.reshape(-1)[:ep]
    ew_q = ew_p.reshape(-1)[:ep]
    ch = 128
    nck = ep // ch
    m0 = (nck * 6 // 10) // NS // 2 * 2
    m1 = nck // NS - m0
    g16 = ch // LANES
    rpt = n_pad // NS

    @functools.partial(
        pl.kernel,
        out_type=jax.ShapeDtypeStruct((NC, n_pad, d), jnp.float32),
        mesh=_sc_mesh(),
        scratch_types=[
            pltpu.VMEM((ch,), jnp.int32),
            pltpu.VMEM((ch,), jnp.int32),
            pltpu.VMEM((ch,), jnp.float32),
            pltpu.VMEM((ch,), jnp.int32),
            pltpu.VMEM((ch,), jnp.int32),
            pltpu.VMEM((ch,), jnp.float32),
            pltpu.VMEM((ch, d), jnp.float32),
            pltpu.VMEM((ch, d), jnp.float32),
            pltpu.VMEM_SHARED((n_pad, d), jnp.float32),
            pltpu.SemaphoreType.DMA,
            pltpu.SemaphoreType.DMA,
            pltpu.SemaphoreType.DMA,
            pltpu.SemaphoreType.DMA,
        ],
    )
    def prop_kernel(src_hbm, dst_hbm, ew_hbm, h_hbm, z_hbm, out_hbm,
                    srcv0, dstv0, ewv0, srcv1, dstv1, ewv1,
                    rows0, rows1, acc, g0, g1, s0, s1):
        cid = lax.axis_index("c")
        sid = lax.axis_index("s")
        start = jnp.where(cid == 0, sid * m0, NS * m0 + sid * m1)
        count = jnp.where(cid == 0, m0, m1)
        pltpu.sync_copy(z_hbm, acc.at[pl.ds(sid * rpt, rpt)])
        plsc.subcore_barrier()

        def scale(rows, ewv):
            def group(g, c2):
                ew_vec = ewv[pl.ds(g * LANES, LANES)]
                for l in range(LANES):
                    lane = jnp.full((LANES, 1), l, jnp.int32)
                    ew_b = lax.gather(
                        ew_vec, lane,
                        lax.GatherDimensionNumbers(
                            offset_dims=(), collapsed_slice_dims=(0,),
                            start_index_map=(0,)),
                        (1,), mode=lax.GatherScatterMode.PROMISE_IN_BOUNDS)
                    row = g * LANES + l
                    for j in range(d // LANES):
                        sl = pl.ds(j * LANES, LANES)
                        rows[row, sl] = rows[row, sl] * ew_b
                return c2
            lax.fori_loop(0, g16, group, 0)

        def load_idx(i, srcv, dstv, ewv):
            base = (start + i) * ch
            pltpu.sync_copy(src_hbm.at[pl.ds(base, ch)], srcv)
            pltpu.sync_copy(dst_hbm.at[pl.ds(base, ch)], dstv)
            pltpu.sync_copy(ew_hbm.at[pl.ds(base, ch)], ewv)

        load_idx(0, srcv0, dstv0, ewv0)
        pltpu.async_copy(h_hbm.at[srcv0], rows0, g0)

        def pair(k, carry):
            a = 2 * k
            b = a + 1

            @pl.when(k > 0)
            def _():
                pltpu.make_async_copy(rows1, acc.at[dstv1], s1).wait()

            load_idx(b, srcv1, dstv1, ewv1)
            pltpu.async_copy(h_hbm.at[srcv1], rows1, g1)
            pltpu.make_async_copy(h_hbm.at[srcv0], rows0, g0).wait()
            scale(rows0, ewv0)
            pltpu.async_copy(rows0, acc.at[dstv0], s0, add=True)
            pltpu.make_async_copy(h_hbm.at[srcv1], rows1, g1).wait()
            scale(rows1, ewv1)
            pltpu.async_copy(rows1, acc.at[dstv1], s1, add=True)

            @pl.when(k < count // 2 - 1)
            def _():
                pltpu.make_async_copy(rows0, acc.at[dstv0], s0).wait()
                load_idx(a + 2, srcv0, dstv0, ewv0)
                pltpu.async_copy(h_hbm.at[srcv0], rows0, g0)

            return carry

        lax.fori_loop(0, count // 2, pair, 0)
        pltpu.make_async_copy(rows0, acc.at[dstv0], s0).wait()
        pltpu.make_async_copy(rows1, acc.at[dstv1], s1).wait()
        plsc.subcore_barrier()
        pltpu.sync_copy(acc.at[pl.ds(sid * rpt, rpt)],
                        out_hbm.at[cid, pl.ds(sid * rpt, rpt)])

    return prop_kernel(src_p, dst_p, ew_q, h, z_blk)


def _tc_first(x, w1, deg0, deg1, bn):
    n, f = x.shape
    d = w1.shape[1]
    grid = n // bn

    def body(x_ref, w_ref, d0_ref, d1_ref, h_ref, dinv_ref):
        deg = d0_ref[...] + d1_ref[...] + 1.0
        dinv = lax.rsqrt(deg)
        dinv_ref[...] = dinv
        h = jnp.dot(x_ref[...], w_ref[...], preferred_element_type=jnp.float32)
        h_ref[...] = h * dinv

    return pl.pallas_call(
        body,
        grid=(grid,),
        in_specs=[
            pl.BlockSpec((bn, f), lambda i: (i, 0)),
            pl.BlockSpec((f, d), lambda i: (0, 0)),
            pl.BlockSpec((bn, 1), lambda i: (i, 0)),
            pl.BlockSpec((bn, 1), lambda i: (i, 0)),
        ],
        out_specs=[
            pl.BlockSpec((bn, d), lambda i: (i, 0)),
            pl.BlockSpec((bn, 1), lambda i: (i, 0)),
        ],
        out_shape=[
            jax.ShapeDtypeStruct((n, d), jnp.float32),
            jax.ShapeDtypeStruct((n, 1), jnp.float32),
        ],
    )(x, w1, deg0, deg1)


def _tc_combine_matmul(acc, hs, dinv, b_row, w, bn):
    n, d = hs.shape
    d2 = w.shape[1]
    grid = n // bn

    def body(a_ref, h_ref, dinv_ref, b_ref, w_ref, o_ref):
        dinv = dinv_ref[...]
        z = dinv * (a_ref[0] + a_ref[1] + h_ref[...]) + b_ref[...]
        a = jnp.maximum(z, 0.0)
        o_ref[...] = dinv * jnp.dot(a, w_ref[...],
                                    preferred_element_type=jnp.float32)

    return pl.pallas_call(
        body,
        grid=(grid,),
        in_specs=[
            pl.BlockSpec((NC, bn, d), lambda i: (0, i, 0)),
            pl.BlockSpec((bn, d), lambda i: (i, 0)),
            pl.BlockSpec((bn, 1), lambda i: (i, 0)),
            pl.BlockSpec((1, d), lambda i: (0, 0)),
            pl.BlockSpec((d, d2), lambda i: (0, 0)),
        ],
        out_specs=pl.BlockSpec((bn, d2), lambda i: (i, 0)),
        out_shape=jax.ShapeDtypeStruct((n, d2), jnp.float32),
    )(acc, hs, dinv, b_row, w)


def _tc_pool_mlp(acc, hs, dinv, b_row, batch3, wl1, bl1_row, wl2, bl2_row,
                 n_graphs, bn):
    n, d = hs.shape
    grid = n // bn
    dm = wl1.shape[1]
    c = wl2.shape[1]

    def body(a_ref, h_ref, dinv_ref, b_ref, bt_ref, wl1_ref, bl1_ref,
             wl2_ref, bl2_ref, o_ref, sums, cnts):
        i = pl.program_id(0)
        z = dinv_ref[...] * (a_ref[0] + a_ref[1] + h_ref[...]) + b_ref[...]
        a = jnp.maximum(z, 0.0)
        bt = bt_ref[...].reshape(1, bn)
        gids = lax.broadcasted_iota(jnp.int32, (n_graphs, bn), 0)
        p = jnp.where(gids == bt, 1.0, 0.0)

        @pl.when(i == 0)
        def _():
            sums[...] = jnp.zeros_like(sums)
            cnts[...] = jnp.zeros_like(cnts)

        sums[...] += jnp.dot(p, a, preferred_element_type=jnp.float32)
        cnts[...] += jnp.sum(p, axis=1, keepdims=True)

        @pl.when(i == grid - 1)
        def _():
            pooled = sums[...] / jnp.maximum(cnts[...], 1.0)
            hm = jnp.maximum(
                jnp.dot(pooled, wl1_ref[...],
                        preferred_element_type=jnp.float32) + bl1_ref[...], 0.0)
            o_ref[...] = jnp.dot(hm, wl2_ref[...],
                                 preferred_element_type=jnp.float32) + bl2_ref[...]

    return pl.pallas_call(
        body,
        grid=(grid,),
        in_specs=[
            pl.BlockSpec((NC, bn, d), lambda i: (0, i, 0)),
            pl.BlockSpec((bn, d), lambda i: (i, 0)),
            pl.BlockSpec((bn, 1), lambda i: (i, 0)),
            pl.BlockSpec((1, d), lambda i: (0, 0)),
            pl.BlockSpec((1, 1, bn), lambda i: (i, 0, 0)),
            pl.BlockSpec((d, dm), lambda i: (0, 0)),
            pl.BlockSpec((1, dm), lambda i: (0, 0)),
            pl.BlockSpec((dm, c), lambda i: (0, 0)),
            pl.BlockSpec((1, c), lambda i: (0, 0)),
        ],
        out_specs=pl.BlockSpec((n_graphs, c), lambda i: (0, 0)),
        out_shape=jax.ShapeDtypeStruct((n_graphs, c), jnp.float32),
        scratch_shapes=[
            pltpu.VMEM((n_graphs, d), jnp.float32),
            pltpu.VMEM((n_graphs, 1), jnp.float32),
        ],
    )(acc, hs, dinv, b_row, batch3, wl1, bl1_row, wl2, bl2_row)


def kernel(x, edge_index, edge_attr, batch, W1, b1, W2, b2, W3, b3,
           Wl1, bl1, Wl2, bl2):
    n, f = x.shape
    e = edge_index.shape[1]
    g = 64
    bn = 1000
    n_pad = 10240
    ep = 163840

    d_all = W2.shape[1]

    src = edge_index[0].astype(jnp.int32)
    dst = edge_index[1].astype(jnp.int32)
    ew = edge_attr.astype(jnp.float32)
    pad = ep - e
    src_p = jnp.concatenate([src, jnp.zeros((pad,), jnp.int32)])
    dst_p = jnp.concatenate([dst, jnp.zeros((pad,), jnp.int32)])
    ew_p = jnp.concatenate([ew, jnp.zeros((pad,), jnp.float32)])

    w1p = jnp.pad(W1, ((0, 0), (0, d_all - W1.shape[1])))
    b1p = jnp.pad(b1, (0, d_all - b1.shape[0]))
    w2p = jnp.pad(W2, ((0, d_all - W2.shape[0]), (0, 0)))

    z_col = jnp.zeros((n_pad // NS,), jnp.float32)
    z128 = jnp.zeros((n_pad // NS, d_all), jnp.float32)

    ch = 256
    nch = ep // ch
    slop = (nch // NS) * ch
    src3 = jnp.concatenate([src_p, jnp.zeros((slop,), jnp.int32)]
                           ).reshape(-1, 1, ch)
    dst3 = jnp.concatenate([dst_p, jnp.zeros((slop,), jnp.int32)]
                           ).reshape(-1, 1, ch)
    ew3 = jnp.concatenate([ew_p, jnp.zeros((slop,), jnp.float32)]
                          ).reshape(-1, 1, ch)

    deg = _sc_degree(dst_p, ew_p, z_col, n_pad)
    deg0 = deg[0, :n].reshape(n, 1)
    deg1 = deg[1, :n].reshape(n, 1)

    h1s, dinv = _tc_first(x, w1p, deg0, deg1, bn)
    acc1 = _sc_propagate(src3, dst3, ew3, h1s, z128, nch, n_pad, d_all)

    h2s = _tc_combine_matmul(acc1, h1s, dinv, b1p.reshape(1, -1), w2p, bn)
    acc2 = _sc_propagate(src3, dst3, ew3, h2s, z128, nch, n_pad, W2.shape[1])

    h3s = _tc_combine_matmul(acc2, h2s, dinv, b2.reshape(1, -1), W3, bn)
    acc3 = _sc_propagate(src3, dst3, ew3, h3s, z128, nch, n_pad, W3.shape[1])

    batch3 = batch.astype(jnp.int32).reshape(n // bn, 1, bn)
    out = _tc_pool_mlp(acc3, h3s, dinv, b3.reshape(1, -1), batch3,
                       Wl1, bl1.reshape(1, -1), Wl2, bl2.reshape(1, -1), g, bn)
    return out


# speedup vs baseline: 1.3398x; 1.0680x over previous
import functools

import jax
import jax.numpy as jnp
from jax import lax
from jax.experimental import pallas as pl
from jax.experimental.pallas import tpu as pltpu
from jax.experimental.pallas import tpu_sc as plsc

NC, NS, LANES = 2, 16, 16
NW = NC * NS


def _sc_mesh():
    return plsc.VectorSubcoreMesh(
        core_axis_name="c", subcore_axis_name="s", num_cores=NC, num_subcores=NS)


def _sc_degree(dst_p, ew_p, z_col, n_pad):
    ep = dst_p.shape[0]
    pt = ep // NW
    ch = 512
    n_ch = pt // ch
    rpt = n_pad // NS

    @functools.partial(
        pl.kernel,
        out_type=jax.ShapeDtypeStruct((NC, n_pad), jnp.float32),
        mesh=_sc_mesh(),
        scratch_types=[
            pltpu.VMEM((ch,), jnp.int32),
            pltpu.VMEM((ch,), jnp.float32),
            pltpu.VMEM_SHARED((n_pad,), jnp.float32),
        ],
    )
    def deg_kernel(dst_hbm, ew_hbm, z_hbm, out_hbm, dstv, ewv, acc):
        cid = lax.axis_index("c")
        sid = lax.axis_index("s")
        wid = sid * NC + cid
        pltpu.sync_copy(z_hbm, acc.at[pl.ds(sid * rpt, rpt)])
        plsc.subcore_barrier()

        def body(i, carry):
            base = wid * pt + i * ch
            pltpu.sync_copy(dst_hbm.at[pl.ds(base, ch)], dstv)
            pltpu.sync_copy(ew_hbm.at[pl.ds(base, ch)], ewv)
            pltpu.sync_copy(ewv, acc.at[dstv], add=True)
            return carry

        lax.fori_loop(0, n_ch, body, 0)
        plsc.subcore_barrier()
        pltpu.sync_copy(acc.at[pl.ds(sid * rpt, rpt)],
                        out_hbm.at[cid, pl.ds(sid * rpt, rpt)])

    return deg_kernel(dst_p, ew_p, z_col)


def _sc_propagate(src3, dst3, ew_p, h, z_blk, nch, n_pad, d):
    ep = nch * 256
    src_p = src3.reshape(-1)[:ep]
    dst_p = dst3.reshape(-1)[:ep]
    ew_q = ew_p.reshape(-1)[:ep]
    ch = 128
    nck = ep // ch
    m0 = (nck * 7 // 10) // NS // 2 * 2
    m1 = nck // NS - m0
    g16 = ch // LANES
    rpt = n_pad // NS

    @functools.partial(
        pl.kernel,
        out_type=jax.ShapeDtypeStruct((NC, n_pad, d), jnp.float32),
        mesh=_sc_mesh(),
        scratch_types=[
            pltpu.VMEM((ch,), jnp.int32),
            pltpu.VMEM((ch,), jnp.int32),
            pltpu.VMEM((ch,), jnp.float32),
            pltpu.VMEM((ch,), jnp.int32),
            pltpu.VMEM((ch,), jnp.int32),
            pltpu.VMEM((ch,), jnp.float32),
            pltpu.VMEM((ch, d), jnp.float32),
            pltpu.VMEM((ch, d), jnp.float32),
            pltpu.VMEM_SHARED((n_pad, d), jnp.float32),
            pltpu.SemaphoreType.DMA,
            pltpu.SemaphoreType.DMA,
            pltpu.SemaphoreType.DMA,
            pltpu.SemaphoreType.DMA,
        ],
    )
    def prop_kernel(src_hbm, dst_hbm, ew_hbm, h_hbm, z_hbm, out_hbm,
                    srcv0, dstv0, ewv0, srcv1, dstv1, ewv1,
                    rows0, rows1, acc, g0, g1, s0, s1):
        cid = lax.axis_index("c")
        sid = lax.axis_index("s")
        start = jnp.where(cid == 0, sid * m0, NS * m0 + sid * m1)
        count = jnp.where(cid == 0, m0, m1)
        pltpu.sync_copy(z_hbm, acc.at[pl.ds(sid * rpt, rpt)])
        plsc.subcore_barrier()

        def scale(rows, ewv):
            def group(g, c2):
                ew_vec = ewv[pl.ds(g * LANES, LANES)]
                for l in range(LANES):
                    lane = jnp.full((LANES, 1), l, jnp.int32)
                    ew_b = lax.gather(
                        ew_vec, lane,
                        lax.GatherDimensionNumbers(
                            offset_dims=(), collapsed_slice_dims=(0,),
                            start_index_map=(0,)),
                        (1,), mode=lax.GatherScatterMode.PROMISE_IN_BOUNDS)
                    row = g * LANES + l
                    for j in range(d // LANES):
                        sl = pl.ds(j * LANES, LANES)
                        rows[row, sl] = rows[row, sl] * ew_b
                return c2
            lax.fori_loop(0, g16, group, 0)

        def load_idx(i, srcv, dstv, ewv):
            base = (start + i) * ch
            pltpu.sync_copy(src_hbm.at[pl.ds(base, ch)], srcv)
            pltpu.sync_copy(dst_hbm.at[pl.ds(base, ch)], dstv)
            pltpu.sync_copy(ew_hbm.at[pl.ds(base, ch)], ewv)

        load_idx(0, srcv0, dstv0, ewv0)
        pltpu.async_copy(h_hbm.at[srcv0], rows0, g0)

        def pair(k, carry):
            a = 2 * k
            b = a + 1

            @pl.when(k > 0)
            def _():
                pltpu.make_async_copy(rows1, acc.at[dstv1], s1).wait()

            load_idx(b, srcv1, dstv1, ewv1)
            pltpu.async_copy(h_hbm.at[srcv1], rows1, g1)
            pltpu.make_async_copy(h_hbm.at[srcv0], rows0, g0).wait()
            scale(rows0, ewv0)
            pltpu.async_copy(rows0, acc.at[dstv0], s0, add=True)
            pltpu.make_async_copy(h_hbm.at[srcv1], rows1, g1).wait()
            scale(rows1, ewv1)
            pltpu.async_copy(rows1, acc.at[dstv1], s1, add=True)

            @pl.when(k < count // 2 - 1)
            def _():
                pltpu.make_async_copy(rows0, acc.at[dstv0], s0).wait()
                load_idx(a + 2, srcv0, dstv0, ewv0)
                pltpu.async_copy(h_hbm.at[srcv0], rows0, g0)

            return carry

        lax.fori_loop(0, count // 2, pair, 0)
        pltpu.make_async_copy(rows0, acc.at[dstv0], s0).wait()
        pltpu.make_async_copy(rows1, acc.at[dstv1], s1).wait()
        plsc.subcore_barrier()
        pltpu.sync_copy(acc.at[pl.ds(sid * rpt, rpt)],
                        out_hbm.at[cid, pl.ds(sid * rpt, rpt)])

    return prop_kernel(src_p, dst_p, ew_q, h, z_blk)


def _tc_first(x, w1, deg0, deg1, bn):
    n, f = x.shape
    d = w1.shape[1]
    grid = n // bn

    def body(x_ref, w_ref, d0_ref, d1_ref, h_ref, dinv_ref):
        deg = d0_ref[...] + d1_ref[...] + 1.0
        dinv = lax.rsqrt(deg)
        dinv_ref[...] = dinv
        h = jnp.dot(x_ref[...], w_ref[...], preferred_element_type=jnp.float32)
        h_ref[...] = h * dinv

    return pl.pallas_call(
        body,
        grid=(grid,),
        in_specs=[
            pl.BlockSpec((bn, f), lambda i: (i, 0)),
            pl.BlockSpec((f, d), lambda i: (0, 0)),
            pl.BlockSpec((bn, 1), lambda i: (i, 0)),
            pl.BlockSpec((bn, 1), lambda i: (i, 0)),
        ],
        out_specs=[
            pl.BlockSpec((bn, d), lambda i: (i, 0)),
            pl.BlockSpec((bn, 1), lambda i: (i, 0)),
        ],
        out_shape=[
            jax.ShapeDtypeStruct((n, d), jnp.float32),
            jax.ShapeDtypeStruct((n, 1), jnp.float32),
        ],
    )(x, w1, deg0, deg1)


def _tc_combine_matmul(acc, hs, dinv, b_row, w, bn):
    n, d = hs.shape
    d2 = w.shape[1]
    grid = n // bn

    def body(a_ref, h_ref, dinv_ref, b_ref, w_ref, o_ref):
        dinv = dinv_ref[...]
        z = dinv * (a_ref[0] + a_ref[1] + h_ref[...]) + b_ref[...]
        a = jnp.maximum(z, 0.0)
        o_ref[...] = dinv * jnp.dot(a, w_ref[...],
                                    preferred_element_type=jnp.float32)

    return pl.pallas_call(
        body,
        grid=(grid,),
        in_specs=[
            pl.BlockSpec((NC, bn, d), lambda i: (0, i, 0)),
            pl.BlockSpec((bn, d), lambda i: (i, 0)),
            pl.BlockSpec((bn, 1), lambda i: (i, 0)),
            pl.BlockSpec((1, d), lambda i: (0, 0)),
            pl.BlockSpec((d, d2), lambda i: (0, 0)),
        ],
        out_specs=pl.BlockSpec((bn, d2), lambda i: (i, 0)),
        out_shape=jax.ShapeDtypeStruct((n, d2), jnp.float32),
    )(acc, hs, dinv, b_row, w)


def _tc_pool_mlp(acc, hs, dinv, b_row, batch3, wl1, bl1_row, wl2, bl2_row,
                 n_graphs, bn):
    n, d = hs.shape
    grid = n // bn
    dm = wl1.shape[1]
    c = wl2.shape[1]

    def body(a_ref, h_ref, dinv_ref, b_ref, bt_ref, wl1_ref, bl1_ref,
             wl2_ref, bl2_ref, o_ref, sums, cnts):
        i = pl.program_id(0)
        z = dinv_ref[...] * (a_ref[0] + a_ref[1] + h_ref[...]) + b_ref[...]
        a = jnp.maximum(z, 0.0)
        bt = bt_ref[...].reshape(1, bn)
        gids = lax.broadcasted_iota(jnp.int32, (n_graphs, bn), 0)
        p = jnp.where(gids == bt, 1.0, 0.0)

        @pl.when(i == 0)
        def _():
            sums[...] = jnp.zeros_like(sums)
            cnts[...] = jnp.zeros_like(cnts)

        sums[...] += jnp.dot(p, a, preferred_element_type=jnp.float32)
        cnts[...] += jnp.sum(p, axis=1, keepdims=True)

        @pl.when(i == grid - 1)
        def _():
            pooled = sums[...] / jnp.maximum(cnts[...], 1.0)
            hm = jnp.maximum(
                jnp.dot(pooled, wl1_ref[...],
                        preferred_element_type=jnp.float32) + bl1_ref[...], 0.0)
            o_ref[...] = jnp.dot(hm, wl2_ref[...],
                                 preferred_element_type=jnp.float32) + bl2_ref[...]

    return pl.pallas_call(
        body,
        grid=(grid,),
        in_specs=[
            pl.BlockSpec((NC, bn, d), lambda i: (0, i, 0)),
            pl.BlockSpec((bn, d), lambda i: (i, 0)),
            pl.BlockSpec((bn, 1), lambda i: (i, 0)),
            pl.BlockSpec((1, d), lambda i: (0, 0)),
            pl.BlockSpec((1, 1, bn), lambda i: (i, 0, 0)),
            pl.BlockSpec((d, dm), lambda i: (0, 0)),
            pl.BlockSpec((1, dm), lambda i: (0, 0)),
            pl.BlockSpec((dm, c), lambda i: (0, 0)),
            pl.BlockSpec((1, c), lambda i: (0, 0)),
        ],
        out_specs=pl.BlockSpec((n_graphs, c), lambda i: (0, 0)),
        out_shape=jax.ShapeDtypeStruct((n_graphs, c), jnp.float32),
        scratch_shapes=[
            pltpu.VMEM((n_graphs, d), jnp.float32),
            pltpu.VMEM((n_graphs, 1), jnp.float32),
        ],
    )(acc, hs, dinv, b_row, batch3, wl1, bl1_row, wl2, bl2_row)


def kernel(x, edge_index, edge_attr, batch, W1, b1, W2, b2, W3, b3,
           Wl1, bl1, Wl2, bl2):
    n, f = x.shape
    e = edge_index.shape[1]
    g = 64
    bn = 1000
    n_pad = 10240
    ep = 163840

    d_all = W2.shape[1]

    src = edge_index[0].astype(jnp.int32)
    dst = edge_index[1].astype(jnp.int32)
    ew = edge_attr.astype(jnp.float32)
    pad = ep - e
    src_p = jnp.concatenate([src, jnp.zeros((pad,), jnp.int32)])
    dst_p = jnp.concatenate([dst, jnp.zeros((pad,), jnp.int32)])
    ew_p = jnp.concatenate([ew, jnp.zeros((pad,), jnp.float32)])

    w1p = jnp.pad(W1, ((0, 0), (0, d_all - W1.shape[1])))
    b1p = jnp.pad(b1, (0, d_all - b1.shape[0]))
    w2p = jnp.pad(W2, ((0, d_all - W2.shape[0]), (0, 0)))

    z_col = jnp.zeros((n_pad // NS,), jnp.float32)
    z128 = jnp.zeros((n_pad // NS, d_all), jnp.float32)

    ch = 256
    nch = ep // ch
    slop = (nch // NS) * ch
    src3 = jnp.concatenate([src_p, jnp.zeros((slop,), jnp.int32)]
                           ).reshape(-1, 1, ch)
    dst3 = jnp.concatenate([dst_p, jnp.zeros((slop,), jnp.int32)]
                           ).reshape(-1, 1, ch)
    ew3 = jnp.concatenate([ew_p, jnp.zeros((slop,), jnp.float32)]
                          ).reshape(-1, 1, ch)

    deg = _sc_degree(dst_p, ew_p, z_col, n_pad)
    deg0 = deg[0, :n].reshape(n, 1)
    deg1 = deg[1, :n].reshape(n, 1)

    h1s, dinv = _tc_first(x, w1p, deg0, deg1, bn)
    acc1 = _sc_propagate(src3, dst3, ew3, h1s, z128, nch, n_pad, d_all)

    h2s = _tc_combine_matmul(acc1, h1s, dinv, b1p.reshape(1, -1), w2p, bn)
    acc2 = _sc_propagate(src3, dst3, ew3, h2s, z128, nch, n_pad, W2.shape[1])

    h3s = _tc_combine_matmul(acc2, h2s, dinv, b2.reshape(1, -1), W3, bn)
    acc3 = _sc_propagate(src3, dst3, ew3, h3s, z128, nch, n_pad, W3.shape[1])

    batch3 = batch.astype(jnp.int32).reshape(n // bn, 1, bn)
    out = _tc_pool_mlp(acc3, h3s, dinv, b3.reshape(1, -1), batch3,
                       Wl1, bl1.reshape(1, -1), Wl2, bl2.reshape(1, -1), g, bn)
    return out
